# speedup vs baseline: 2.5454x; 2.5454x over previous
import functools
import math

import jax
import jax.numpy as jnp
from jax import lax
from jax.experimental import pallas as pl
from jax.experimental.pallas import tpu as pltpu
from jax.experimental.pallas import tpu_sc as plsc

D = 128
S = 200
B = 1024
N = B * S
NC = 2
NS = 16
NW = NC * NS
PER_W = N // NW
R = 128
CHUNKS = PER_W // R
LANES = 16
SCALE = math.sqrt(float(D))

_mesh = plsc.VectorSubcoreMesh(core_axis_name="c", subcore_axis_name="s")


@functools.partial(
    pl.kernel,
    mesh=_mesh,
    out_type=jax.ShapeDtypeStruct((N, D), jnp.float32),
    scratch_types=[
        pltpu.VMEM((CHUNKS, R), jnp.int32),
        pltpu.VMEM((S, D), jnp.float32),
        pltpu.VMEM((R, D), jnp.float32),
        pltpu.VMEM((R, D), jnp.float32),
        pltpu.SemaphoreType.DMA,
        pltpu.SemaphoreType.DMA,
    ],
)
def _emb_kernel(idx_hbm, table_hbm, pe_hbm, out_hbm,
                idx_v, pe_v, buf0, buf1, sem0, sem1):
    wid = lax.axis_index("s") * NC + lax.axis_index("c")
    base = wid * PER_W

    pltpu.sync_copy(idx_hbm.at[wid], idx_v)
    pltpu.sync_copy(pe_hbm.at[pl.ds(0, S)], pe_v)

    pltpu.async_copy(table_hbm.at[idx_v.at[0]], buf0, sem0)

    def chunk_step(c, buf, sem, nbuf, nsem):
        pltpu.make_async_copy(table_hbm.at[idx_v.at[c]], buf, sem).wait()

        @pl.when(c + 1 < CHUNKS)
        def _():
            pltpu.async_copy(table_hbm.at[idx_v.at[c + 1]], nbuf, nsem)

        pos0 = lax.rem(c * R, S)

        def row_body(r, pos):
            for j in range(D // LANES):
                sl = pl.ds(j * LANES, LANES)
                buf[r, sl] = buf[r, sl] * SCALE + pe_v[pos, sl]
            pos = pos + 1
            return lax.select(pos == S, 0, pos)

        lax.fori_loop(0, R, row_body, pos0)

        pltpu.sync_copy(buf, out_hbm.at[pl.ds(base + c * R, R)])

    def outer(i, carry):
        c0 = i * 2
        chunk_step(c0, buf0, sem0, buf1, sem1)
        chunk_step(c0 + 1, buf1, sem1, buf0, sem0)
        return carry

    lax.fori_loop(0, CHUNKS // 2, outer, 0)


def kernel(x, table, pe):
    idx = jnp.reshape(x, (NW, CHUNKS, R))
    out = _emb_kernel(idx, table, pe)
    return jnp.reshape(out, (B, S, D))


# speedup vs baseline: 6.2618x; 2.4600x over previous
import functools
import math

import jax
import jax.numpy as jnp
from jax import lax
from jax.experimental import pallas as pl
from jax.experimental.pallas import tpu as pltpu
from jax.experimental.pallas import tpu_sc as plsc

D = 128
S = 200
B = 1024
N = B * S
NC = 2
NS = 16
NW = NC * NS
PER_W = N // NW
R = 128
CHUNKS = PER_W // R
LANES = 16
SCALE = math.sqrt(float(D))

_mesh = plsc.VectorSubcoreMesh(core_axis_name="c", subcore_axis_name="s")


@functools.partial(
    pl.kernel,
    mesh=_mesh,
    out_type=jax.ShapeDtypeStruct((N, D), jnp.float32),
    scratch_types=[
        pltpu.VMEM((CHUNKS, R), jnp.int32),
        pltpu.VMEM((S, D), jnp.float32),
        pltpu.VMEM((R, D), jnp.float32),
        pltpu.VMEM((R, D), jnp.float32),
        pltpu.SemaphoreType.DMA,
        pltpu.SemaphoreType.DMA,
    ],
)
def _emb_kernel(idx_hbm, table_hbm, pe_hbm, out_hbm,
                idx_v, pe_v, buf0, buf1, sem0, sem1):
    wid = lax.axis_index("s") * NC + lax.axis_index("c")
    base = wid * PER_W

    pltpu.sync_copy(idx_hbm.at[wid], idx_v)
    pltpu.sync_copy(pe_hbm.at[pl.ds(0, S)], pe_v)

    pltpu.async_copy(table_hbm.at[idx_v.at[0]], buf0, sem0)

    def chunk_step(c, buf, sem, nbuf, nsem):
        pltpu.make_async_copy(table_hbm.at[idx_v.at[c]], buf, sem).wait()

        @pl.when(c + 1 < CHUNKS)
        def _():
            pltpu.async_copy(table_hbm.at[idx_v.at[c + 1]], nbuf, nsem)

        pos0 = lax.rem(c * R, S)

        @plsc.parallel_loop(0, R, 1, unroll=4)
        def row_body(r):
            pos = pos0 + r
            pos = lax.select(pos >= S, pos - S, pos)
            for j in range(D // LANES):
                sl = pl.ds(j * LANES, LANES)
                buf[r, sl] = buf[r, sl] * SCALE + pe_v[pos, sl]

        pltpu.sync_copy(buf, out_hbm.at[pl.ds(base + c * R, R)])

    def outer(i, carry):
        c0 = i * 2
        chunk_step(c0, buf0, sem0, buf1, sem1)
        chunk_step(c0 + 1, buf1, sem1, buf0, sem0)
        return carry

    lax.fori_loop(0, CHUNKS // 2, outer, 0)


def kernel(x, table, pe):
    idx = jnp.reshape(x, (NW, CHUNKS, R))
    out = _emb_kernel(idx, table, pe)
    return jnp.reshape(out, (B, S, D))


# speedup vs baseline: 7.3666x; 1.1764x over previous
import functools
import math

import jax
import jax.numpy as jnp
from jax import lax
from jax.experimental import pallas as pl
from jax.experimental.pallas import tpu as pltpu
from jax.experimental.pallas import tpu_sc as plsc

D = 128
S = 200
B = 1024
N = B * S
NC = 2
NS = 16
NW = NC * NS
PER_W = N // NW
R = 128
CHUNKS = PER_W // R
NB = 5
LANES = 16
SCALE = math.sqrt(float(D))

_mesh = plsc.VectorSubcoreMesh(core_axis_name="c", subcore_axis_name="s")


@functools.partial(
    pl.kernel,
    mesh=_mesh,
    out_type=jax.ShapeDtypeStruct((N, D), jnp.float32),
    scratch_types=[
        pltpu.VMEM((CHUNKS, R), jnp.int32),
        pltpu.VMEM((S, D), jnp.float32),
    ]
    + [pltpu.VMEM((R, D), jnp.float32) for _ in range(NB)]
    + [pltpu.SemaphoreType.DMA for _ in range(2 * NB)],
)
def _emb_kernel(idx_hbm, table_hbm, pe_hbm, out_hbm, idx_v, pe_v, *rest):
    bufs = rest[:NB]
    gsems = rest[NB:2 * NB]
    wsems = rest[2 * NB:]

    wid = lax.axis_index("s") * NC + lax.axis_index("c")
    base = wid * PER_W

    pltpu.sync_copy(idx_hbm.at[wid], idx_v)
    pltpu.sync_copy(pe_hbm.at[pl.ds(0, S)], pe_v)

    def gather(c, b):
        pltpu.async_copy(table_hbm.at[idx_v.at[c]], bufs[b], gsems[b])

    def wait_gather(c, b):
        pltpu.make_async_copy(table_hbm.at[idx_v.at[c]], bufs[b],
                              gsems[b]).wait()

    def write(c, b):
        pltpu.async_copy(bufs[b], out_hbm.at[pl.ds(base + c * R, R)],
                         wsems[b])

    def wait_write(c, b):
        pltpu.make_async_copy(bufs[b], out_hbm.at[pl.ds(base + c * R, R)],
                              wsems[b]).wait()

    for k in range(NB):
        gather(k, k)

    def step(c, b):
        wait_gather(c, b)

        pos0 = lax.rem(c * R, S)

        @plsc.parallel_loop(0, R, 1, unroll=4)
        def row_body(r):
            pos = pos0 + r
            pos = lax.select(pos >= S, pos - S, pos)
            for j in range(D // LANES):
                sl = pl.ds(j * LANES, LANES)
                bufs[b][r, sl] = bufs[b][r, sl] * SCALE + pe_v[pos, sl]

        write(c, b)

        bn = (b + 3) % NB

        @pl.when((c >= NB - 3) & (c + 3 < CHUNKS))
        def _():
            wait_write(c - (NB - 3), bn)
            gather(c + 3, bn)

    def outer(i, carry):
        for k in range(NB):
            step(i * NB + k, k)
        return carry

    lax.fori_loop(0, CHUNKS // NB, outer, 0)

    for k in range(NB):
        c = CHUNKS - NB + k
        wait_write(c, c % NB)


def kernel(x, table, pe):
    idx = jnp.reshape(x, (NW, CHUNKS, R))
    out = _emb_kernel(idx, table, pe)
    return jnp.reshape(out, (B, S, D))
